# Initial kernel scaffold; baseline (speedup 1.0000x reference)
import jax
import jax.numpy as jnp
from jax.experimental import pallas as pl


def kernel(x, pos, edge_index, edge_attr, pos_high, edge_index_high, edge_attr_high, W1, b1, alpha1, W5, b5, alpha5):
    raise NotImplementedError("write your pallas kernel here")


# speedup vs baseline: 2.9113x; 2.9113x over previous
import functools

import jax
import jax.numpy as jnp
from jax import lax
from jax.experimental import pallas as pl
from jax.experimental.pallas import tpu as pltpu
from jax.experimental.pallas import tpu_sc as plsc

N = 10000
E = 320000
F = 128
K = 4
POSD = 3
KNN_K = 50

BN = 1000
BT = 80
EPT = E // 16
NPAD = 10240
NPT = NPAD // 16


def _proj_body(m_ref, wm_ref, wp_ref, b_ref, a_ref, p_ref):
    m = m_ref[...]
    k = pl.program_id(0)
    a_ref[0] = jnp.dot(m, wm_ref[0], preferred_element_type=jnp.float32)
    p_ref[0] = (jnp.dot(m[:, F:], wp_ref[0], preferred_element_type=jnp.float32)
                + b_ref[k][None, :])


def _project(m, wm, wp, b):
    return pl.pallas_call(
        _proj_body,
        grid=(K, N // BN),
        in_specs=[
            pl.BlockSpec((BN, F + POSD), lambda k, i: (i, 0)),
            pl.BlockSpec((1, F + POSD, F), lambda k, i: (k, 0, 0)),
            pl.BlockSpec((1, POSD, F), lambda k, i: (k, 0, 0)),
            pl.BlockSpec((K, F), lambda k, i: (0, 0)),
        ],
        out_specs=[
            pl.BlockSpec((1, BN, F), lambda k, i: (k, i, 0)),
            pl.BlockSpec((1, BN, F), lambda k, i: (k, i, 0)),
        ],
        out_shape=[jax.ShapeDtypeStruct((K, N, F), jnp.float32)] * 2,
    )(m, wm, wp, b)


def _comb_e(s_ref, deg_ref, amat_ref):
    deg = jnp.maximum(deg_ref[...], 1.0)
    num = s_ref[0] * amat_ref[0][None, :]
    for k in range(1, K):
        num = num + s_ref[k] * amat_ref[k][None, :]
    return num / deg


def _comb_proj_body(s_ref, deg_ref, amat_ref, pos_ref, wm_ref, wp_ref, b_ref,
                    a_ref, p_ref):
    e = _comb_e(s_ref, deg_ref, amat_ref)
    m = jnp.concatenate([e, pos_ref[...]], axis=1)
    for k in range(K):
        a_ref[k] = jnp.dot(m, wm_ref[k], preferred_element_type=jnp.float32)
        p_ref[k] = (jnp.dot(m[:, F:], wp_ref[k], preferred_element_type=jnp.float32)
                    + b_ref[k][None, :])


def _comb_proj(s, deg, amat, pos, wm, wp, b):
    return pl.pallas_call(
        _comb_proj_body,
        grid=(N // BN,),
        in_specs=[
            pl.BlockSpec((K, BN, F), lambda i: (0, i, 0)),
            pl.BlockSpec((BN, 1), lambda i: (i, 0)),
            pl.BlockSpec((K, F), lambda i: (0, 0)),
            pl.BlockSpec((BN, POSD), lambda i: (i, 0)),
            pl.BlockSpec((K, F + POSD, F), lambda i: (0, 0, 0)),
            pl.BlockSpec((K, POSD, F), lambda i: (0, 0, 0)),
            pl.BlockSpec((K, F), lambda i: (0, 0)),
        ],
        out_specs=[
            pl.BlockSpec((K, BN, F), lambda i: (0, i, 0)),
            pl.BlockSpec((K, BN, F), lambda i: (0, i, 0)),
        ],
        out_shape=[jax.ShapeDtypeStruct((K, N, F), jnp.float32)] * 2,
    )(s, deg, amat, pos, wm, wp, b)


def _comb_body(s_ref, deg_ref, amat_ref, e_ref):
    e_ref[...] = _comb_e(s_ref, deg_ref, amat_ref)


def _combine(s, deg, amat):
    return pl.pallas_call(
        _comb_body,
        grid=(N // BN,),
        in_specs=[
            pl.BlockSpec((K, BN, F), lambda i: (0, i, 0)),
            pl.BlockSpec((BN, 1), lambda i: (i, 0)),
            pl.BlockSpec((K, F), lambda i: (0, 0)),
        ],
        out_specs=pl.BlockSpec((BN, F), lambda i: (i, 0)),
        out_shape=jax.ShapeDtypeStruct((N, F), jnp.float32),
    )(s, deg, amat)


def _sc_conv_body(a_hbm, p_hbm, src_hbm, dst_hbm, s_out, deg_out,
                  acc, degacc, idx_s, idx_d, idx_a, idx_p,
                  rows_a, rows_p, ones_v, zbuf, zbuf1, sem1, sem2):
    cid = lax.axis_index("c")
    sid = lax.axis_index("s")
    ebase = sid * EPT
    rbase = sid * NPT

    zero16 = jnp.zeros((16,), jnp.float32)
    one16 = jnp.full((16,), 1.0, jnp.float32)

    def _zrow(r, c_):
        for c in range(F // 16):
            zbuf[r, pl.ds(c * 16, 16)] = zero16
        return c_
    lax.fori_loop(0, 128, _zrow, jnp.int32(0))

    def _zrow1(r, c_):
        zbuf1[pl.ds(r * 16, 16)] = zero16
        return c_
    lax.fori_loop(0, NPT // 16, _zrow1, jnp.int32(0))

    for v in range(BT // 16):
        ones_v[pl.ds(v * 16, 16)] = one16

    for j in range(NPT // 128):
        pltpu.sync_copy(zbuf, acc.at[pl.ds(rbase + j * 128, 128)])
    pltpu.sync_copy(zbuf1, degacc.at[pl.ds(rbase, NPT)])
    plsc.subcore_barrier()

    for kk in range(2):
        koff = (cid * 2 + kk) * N
        soff = (cid * 2 + kk) * NPAD

        def _batch(b, c_):
            off = ebase + b * BT
            pltpu.sync_copy(src_hbm.at[pl.ds(off, BT)], idx_s)
            pltpu.sync_copy(dst_hbm.at[pl.ds(off, BT)], idx_d)
            koffv = jnp.full((16,), koff, jnp.int32)
            for v in range(BT // 16):
                sl = pl.ds(v * 16, 16)
                idx_a[sl] = idx_s[sl] + koffv
                idx_p[sl] = idx_d[sl] + koffv
            pltpu.async_copy(a_hbm.at[idx_a], rows_a, sem1).wait()
            pltpu.async_copy(p_hbm.at[idx_p], rows_p, sem2).wait()

            def _row(r, cc_):
                for c in range(F // 16):
                    sl = pl.ds(c * 16, 16)
                    z = rows_a[r, sl] + rows_p[r, sl]
                    rows_a[r, sl] = jnp.maximum(z, 0.0) + 0.1 * jnp.minimum(z, 0.0)
                return cc_
            lax.fori_loop(0, BT, _row, jnp.int32(0))

            pltpu.sync_copy(rows_a, acc.at[idx_d], add=True)
            if kk == 0:
                pltpu.sync_copy(ones_v, degacc.at[idx_d], add=True)
            return c_

        lax.fori_loop(0, EPT // BT, _batch, jnp.int32(0))
        plsc.subcore_barrier()

        pltpu.sync_copy(acc.at[pl.ds(rbase, NPT)],
                        s_out.at[pl.ds(soff + rbase, NPT)])
        if kk == 0:
            @pl.when(cid == 0)
            def _():
                pltpu.sync_copy(degacc.at[pl.ds(rbase, NPT)],
                                deg_out.at[pl.ds(rbase, NPT)])
            for j in range(NPT // 128):
                pltpu.sync_copy(zbuf, acc.at[pl.ds(rbase + j * 128, 128)])
            plsc.subcore_barrier()


_SC_MESH = plsc.VectorSubcoreMesh(core_axis_name="c", subcore_axis_name="s")

_sc_conv = pl.kernel(
    _sc_conv_body,
    out_type=[jax.ShapeDtypeStruct((K * NPAD, F), jnp.float32),
              jax.ShapeDtypeStruct((NPAD,), jnp.float32)],
    mesh=_SC_MESH,
    scratch_types=[
        pltpu.VMEM_SHARED((NPAD, F), jnp.float32),
        pltpu.VMEM_SHARED((NPAD,), jnp.float32),
        pltpu.VMEM((BT,), jnp.int32),
        pltpu.VMEM((BT,), jnp.int32),
        pltpu.VMEM((BT,), jnp.int32),
        pltpu.VMEM((BT,), jnp.int32),
        pltpu.VMEM((BT, F), jnp.float32),
        pltpu.VMEM((BT, F), jnp.float32),
        pltpu.VMEM((BT,), jnp.float32),
        pltpu.VMEM((128, F), jnp.float32),
        pltpu.VMEM((NPT,), jnp.float32),
        pltpu.SemaphoreType.DMA,
        pltpu.SemaphoreType.DMA,
    ],
)


def _knn_interp(feat, pos_x, pos_y):
    chunks = pos_y.reshape(-1, 1000, pos_y.shape[-1])

    def cf(pc):
        d2 = jnp.sum((pc[:, None, :] - pos_x[None, :, :]) ** 2, axis=-1)
        negd, idx = jax.lax.top_k(-d2, KNN_K)
        w = 1.0 / (-negd + 1e-16)
        w = w / jnp.sum(w, axis=-1, keepdims=True)
        return jnp.sum(feat[idx] * w[..., None], axis=1)

    out = jax.lax.map(cf, chunks)
    return out.reshape(-1, feat.shape[-1])


def kernel(x, pos, edge_index, edge_attr, pos_high, edge_index_high,
           edge_attr_high, W1, b1, alpha1, W5, b5, alpha5):
    x = x.astype(jnp.float32)
    src = edge_index[0]
    dst = edge_index[1]
    m1 = jnp.concatenate([x, pos], axis=1)

    wm1 = jnp.concatenate([W1[:, :F, :], -W1[:, F:, :]], axis=1)
    a1t, p1t = _project(m1, wm1, W1[:, F:, :], b1)
    s1, deg1 = _sc_conv(a1t.reshape(K * N, F), p1t.reshape(K * N, F), src, dst)
    deg = deg1[:, None]

    a1 = jax.nn.softmax(alpha1)
    a1mat = jnp.broadcast_to(a1[:, None], (K, F))
    wm5 = jnp.concatenate([W5[:, :F, :], -W5[:, F:, :]], axis=1)
    a5t, p5t = _comb_proj(s1.reshape(K, NPAD, F), deg, a1mat, pos,
                          wm5, W5[:, F:, :], b5)
    s5, _ = _sc_conv(a5t.reshape(K * N, F), p5t.reshape(K * N, F), src, dst)

    a5 = jax.nn.softmax(alpha5)
    a5mat = jnp.broadcast_to(a5[:, None], (K, F))
    e5 = _combine(s5.reshape(K, NPAD, F), deg, a5mat)

    return _knn_interp(e5, pos, pos_high)
